# Initial kernel scaffold; baseline (speedup 1.0000x reference)
import functools

import jax
import jax.numpy as jnp
from jax import lax
from jax.experimental import pallas as pl
from jax.experimental.pallas import tpu as pltpu
from jax.experimental.pallas import tpu_sc as plsc

_B, _L = 16, 2048
_CH = 16
_NCH = _L // _CH
_IMAX = jnp.int32(2147483647)


def _body(start_hbm, end_hbm, score_out, i_out, j_out,
          start_v, end_v, s_v, score_s, i_s, j_s):
    w = lax.axis_index("s") * 2 + lax.axis_index("c")

    @pl.when(w < _B)
    def _():
        row = w
        pltpu.sync_copy(start_hbm.at[row], start_v)
        pltpu.sync_copy(end_hbm.at[row], end_v)

        lane = lax.iota(jnp.int32, _CH)
        neg_inf = jnp.float32(-jnp.inf)

        def bwd(k, state):
            carry, best, istar = state
            base = pl.multiple_of((_NCH - 1 - k) * _CH, _CH)
            e = end_v[pl.ds(base, _CH)]
            rc = lax.rev(plsc.cummax(lax.rev(e, (0,))), (0,))
            s_chunk = jnp.maximum(rc, carry)
            s_v[pl.ds(base, _CH)] = s_chunk
            cch = start_v[pl.ds(base, _CH)] + s_chunk
            local = jnp.max(cch)
            idx = lane + base
            local_i = jnp.min(jnp.where(cch == local, idx, _IMAX))
            upd = local >= best
            best = jnp.where(upd, local, best)
            istar = jnp.where(upd, local_i, istar)
            carry = jnp.maximum(carry, jnp.max(e))
            return carry, best, istar

        _, best, istar = lax.fori_loop(
            0, _NCH, bwd, (neg_inf, neg_inf, jnp.int32(0)))

        cb = pl.multiple_of((istar // _CH) * _CH, _CH)
        sv = s_v[pl.ds(cb, _CH)]
        target = jnp.max(jnp.where(lane + cb == istar, sv, neg_inf))

        def fwd(k, jstar):
            base = pl.multiple_of(k * _CH, _CH)
            e = end_v[pl.ds(base, _CH)]
            idx = lane + base
            m = (idx >= istar) & (e == target)
            cand = jnp.min(jnp.where(m, idx, _IMAX))
            return jnp.minimum(jstar, cand)

        jstar = lax.fori_loop(istar // _CH, _NCH, fwd, _IMAX)

        score_s[...] = jnp.zeros((_CH,), jnp.float32) + best
        i_s[...] = jnp.zeros((_CH,), jnp.int32) + istar
        j_s[...] = jnp.zeros((_CH,), jnp.int32) + jstar
        pltpu.sync_copy(score_s, score_out.at[row])
        pltpu.sync_copy(i_s, i_out.at[row])
        pltpu.sync_copy(j_s, j_out.at[row])


_sc_call = functools.partial(
    pl.kernel,
    mesh=plsc.VectorSubcoreMesh(core_axis_name="c", subcore_axis_name="s"),
    out_type=[
        jax.ShapeDtypeStruct((_B, _CH), jnp.float32),
        jax.ShapeDtypeStruct((_B, _CH), jnp.int32),
        jax.ShapeDtypeStruct((_B, _CH), jnp.int32),
    ],
    scratch_types=[
        pltpu.VMEM((_L,), jnp.float32),
        pltpu.VMEM((_L,), jnp.float32),
        pltpu.VMEM((_L,), jnp.float32),
        pltpu.VMEM((_CH,), jnp.float32),
        pltpu.VMEM((_CH,), jnp.int32),
        pltpu.VMEM((_CH,), jnp.int32),
    ],
)(_body)


@jax.jit
def kernel(span_start_logits, span_end_logits):
    score, i_idx, j_idx = _sc_call(span_start_logits, span_end_logits)
    return score[:, 0], i_idx[:, 0], j_idx[:, 0]


# speedup vs baseline: 20.0293x; 20.0293x over previous
import functools

import jax
import jax.numpy as jnp
import numpy as np
from jax import lax
from jax.experimental import pallas as pl
from jax.experimental.pallas import tpu as pltpu
from jax.experimental.pallas import tpu_sc as plsc

_B, _L = 16, 2048
_CH = 16
_NCH = _L // _CH
_IMAX = np.int32(2147483647)


def _body(start_hbm, end_hbm, score_out, i_out, j_out,
          start_v, end_v, s_v, score_s, i_s, j_s):
    w = lax.axis_index("s") * 2 + lax.axis_index("c")

    @pl.when(w < _B)
    def _():
        row = w
        pltpu.sync_copy(start_hbm.at[row], start_v)
        pltpu.sync_copy(end_hbm.at[row], end_v)

        lane = lax.iota(jnp.int32, _CH)
        neg_inf = np.float32(-np.inf)

        def bwd(k, state):
            carry, best, istar = state
            base = pl.multiple_of((_NCH - 1 - k) * _CH, _CH)
            e = end_v[pl.ds(base, _CH)]
            rc = lax.rev(plsc.cummax(lax.rev(e, (0,))), (0,))
            s_chunk = jnp.maximum(rc, carry)
            s_v[pl.ds(base, _CH)] = s_chunk
            cch = start_v[pl.ds(base, _CH)] + s_chunk
            local = jnp.max(cch)
            idx = lane + base
            local_i = jnp.min(jnp.where(cch == local, idx, _IMAX))
            upd = local >= best
            best = jnp.where(upd, local, best)
            istar = jnp.where(upd, local_i, istar)
            carry = jnp.maximum(carry, jnp.max(e))
            return carry, best, istar

        _, best, istar = lax.fori_loop(
            0, _NCH, bwd, (neg_inf, neg_inf, np.int32(0)))

        cb = pl.multiple_of((istar // _CH) * _CH, _CH)
        sv = s_v[pl.ds(cb, _CH)]
        target = jnp.max(jnp.where(lane + cb == istar, sv, neg_inf))

        def fwd(k, jstar):
            base = pl.multiple_of(k * _CH, _CH)
            e = end_v[pl.ds(base, _CH)]
            idx = lane + base
            m = (idx >= istar) & (e == target)
            cand = jnp.min(jnp.where(m, idx, _IMAX))
            return jnp.minimum(jstar, cand)

        jstar = lax.fori_loop(istar // _CH, _NCH, fwd, _IMAX)

        score_s[...] = jnp.zeros((_CH,), jnp.float32) + best
        i_s[...] = jnp.zeros((_CH,), jnp.int32) + istar
        j_s[...] = jnp.zeros((_CH,), jnp.int32) + jstar
        pltpu.sync_copy(score_s, score_out.at[row])
        pltpu.sync_copy(i_s, i_out.at[row])
        pltpu.sync_copy(j_s, j_out.at[row])


_sc_call = functools.partial(
    pl.kernel,
    mesh=plsc.VectorSubcoreMesh(core_axis_name="c", subcore_axis_name="s"),
    compiler_params=pltpu.CompilerParams(needs_layout_passes=False),
    out_type=[
        jax.ShapeDtypeStruct((_B, _CH), jnp.float32),
        jax.ShapeDtypeStruct((_B, _CH), jnp.int32),
        jax.ShapeDtypeStruct((_B, _CH), jnp.int32),
    ],
    scratch_types=[
        pltpu.VMEM((_L,), jnp.float32),
        pltpu.VMEM((_L,), jnp.float32),
        pltpu.VMEM((_L,), jnp.float32),
        pltpu.VMEM((_CH,), jnp.float32),
        pltpu.VMEM((_CH,), jnp.int32),
        pltpu.VMEM((_CH,), jnp.int32),
    ],
)(_body)


@jax.jit
def kernel(span_start_logits, span_end_logits):
    score, i_idx, j_idx = _sc_call(span_start_logits, span_end_logits)
    return score[:, 0], i_idx[:, 0], j_idx[:, 0]
